# baseline (speedup 1.0000x reference)
import functools

import jax
import jax.numpy as jnp
from jax import lax
from jax.experimental import pallas as pl
from jax.experimental.pallas import tpu as pltpu
from jax.experimental.pallas import tpu_sc as plsc

_B = 4096
_D = 128
_LANES = 16
_CHUNKS = _D // _LANES


def _make_loss_kernel(num_workers: int, b_per_w: int):
    mesh = plsc.VectorSubcoreMesh(core_axis_name="c", subcore_axis_name="s")
    half = b_per_w // 2
    idx_w = 3 * b_per_w

    @functools.partial(
        pl.kernel,
        mesh=mesh,
        out_type=jax.ShapeDtypeStruct((num_workers, _LANES), jnp.float32),
        scratch_types=[
            pltpu.VMEM((idx_w,), jnp.int32),
            pltpu.VMEM((b_per_w, _LANES), jnp.float32),
            pltpu.VMEM((b_per_w, _D), jnp.float32),
            pltpu.VMEM((b_per_w, _D), jnp.float32),
            pltpu.VMEM((b_per_w, _D), jnp.float32),
            pltpu.VMEM((_LANES,), jnp.float32),
            pltpu.SemaphoreType.DMA,
            pltpu.SemaphoreType.DMA,
            pltpu.SemaphoreType.DMA,
            pltpu.SemaphoreType.DMA,
        ],
    )
    def loss_kernel(idx_hbm, v_hbm, ent_hbm, rel_hbm, out_hbm,
                    idx_vm, v_vm, h_rows, r_rows, t_rows,
                    acc_vm, sem_idx, sem_v, sem_a, sem_b):
        num_cores = lax.axis_size("c")
        wid = lax.axis_index("s") * num_cores + lax.axis_index("c")

        pltpu.async_copy(idx_hbm.at[wid], idx_vm, sem_idx).wait()

        gathers = []
        for c, sem in ((0, sem_a), (1, sem_b)):
            row_sl = pl.ds(c * half, half)
            gathers.append((
                pltpu.async_copy(
                    ent_hbm.at[idx_vm.at[pl.ds(2 * half * c, half)]],
                    h_rows.at[row_sl], sem),
                pltpu.async_copy(
                    ent_hbm.at[idx_vm.at[pl.ds(2 * half * c + half, half)]],
                    t_rows.at[row_sl], sem),
                pltpu.async_copy(
                    rel_hbm.at[idx_vm.at[pl.ds(2 * b_per_w + c * half, half)]],
                    r_rows.at[row_sl], sem),
            ))
        cp_v = pltpu.async_copy(
            v_hbm.at[pl.ds(wid * b_per_w, b_per_w)], v_vm, sem_v)

        def body(i, acc):
            vv = v_vm[i, :]
            dd = jnp.zeros((_LANES,), jnp.float32)
            for c in range(_CHUNKS):
                sl = pl.ds(c * _LANES, _LANES)
                d = h_rows[i, sl] + r_rows[i, sl] - t_rows[i, sl]
                dd = dd + d * d
            return acc + dd * vv

        cp_v.wait()
        acc = jnp.zeros((_LANES,), jnp.float32)
        for c in range(2):
            for cp in gathers[c]:
                cp.wait()
            acc = lax.fori_loop(c * half, (c + 1) * half, body, acc)

        acc_vm[...] = acc
        pltpu.sync_copy(acc_vm, out_hbm.at[wid])

    return loss_kernel


def kernel(h, r, t, v, adj, ent_emb, rel_emb, W, b):
    info = plsc.get_sparse_core_info()
    num_workers = info.num_cores * info.num_subcores
    b_per_w = _B // num_workers
    half = b_per_w // 2
    loss_kernel = _make_loss_kernel(num_workers, b_per_w)

    h2 = h.astype(jnp.int32).reshape(num_workers, 2, half)
    t2 = t.astype(jnp.int32).reshape(num_workers, 2, half)
    r2 = r.astype(jnp.int32).reshape(num_workers, 2, half)
    ht = jnp.stack([h2, t2], axis=2).reshape(num_workers, 4 * half)
    idx = jnp.concatenate([ht, r2.reshape(num_workers, 2 * half)], axis=1)

    v_rep = jnp.broadcast_to(v.astype(jnp.float32)[:, None], (_B, _LANES))

    partials = loss_kernel(idx, v_rep, ent_emb, rel_emb)
    return jnp.sum(partials) / jnp.float32(_B)


# speedup vs baseline: 1.1038x; 1.1038x over previous
import functools

import jax
import jax.numpy as jnp
from jax import lax
from jax.experimental import pallas as pl
from jax.experimental.pallas import tpu as pltpu
from jax.experimental.pallas import tpu_sc as plsc

_B = 4096
_D = 128
_LANES = 16
_CHUNKS = _D // _LANES


def _make_loss_kernel(num_workers: int, b_per_w: int):
    mesh = plsc.VectorSubcoreMesh(core_axis_name="c", subcore_axis_name="s")
    half = b_per_w // 2

    @functools.partial(
        pl.kernel,
        mesh=mesh,
        out_type=jax.ShapeDtypeStruct((num_workers, _LANES), jnp.float32),
        scratch_types=[
            pltpu.VMEM((b_per_w,), jnp.int32),
            pltpu.VMEM((b_per_w,), jnp.int32),
            pltpu.VMEM((b_per_w,), jnp.int32),
            pltpu.VMEM((b_per_w, _LANES), jnp.float32),
            pltpu.VMEM((b_per_w, _D), jnp.float32),
            pltpu.VMEM((b_per_w, _D), jnp.float32),
            pltpu.VMEM((b_per_w, _D), jnp.float32),
            pltpu.VMEM((_LANES,), jnp.float32),
            pltpu.SemaphoreType.DMA,
            pltpu.SemaphoreType.DMA,
            pltpu.SemaphoreType.DMA,
            pltpu.SemaphoreType.DMA,
        ],
    )
    def loss_kernel(h_hbm, r_hbm, t_hbm, v_hbm, ent_hbm, rel_hbm, out_hbm,
                    h_idx, r_idx, t_idx, v_vm, h_rows, r_rows, t_rows,
                    acc_vm, sem_idx, sem_v, sem_a, sem_b):
        num_cores = lax.axis_size("c")
        wid = lax.axis_index("s") * num_cores + lax.axis_index("c")
        base = wid * b_per_w

        cps = [
            pltpu.async_copy(h_hbm.at[pl.ds(base, b_per_w)], h_idx, sem_idx),
            pltpu.async_copy(r_hbm.at[pl.ds(base, b_per_w)], r_idx, sem_idx),
            pltpu.async_copy(t_hbm.at[pl.ds(base, b_per_w)], t_idx, sem_idx),
        ]
        cp_v = pltpu.async_copy(v_hbm.at[pl.ds(base, b_per_w)], v_vm, sem_v)
        for cp in cps:
            cp.wait()

        halves = []
        for c, sem in ((0, sem_a), (1, sem_b)):
            sl = pl.ds(c * half, half)
            halves.append((
                pltpu.async_copy(ent_hbm.at[h_idx.at[sl]],
                                 h_rows.at[sl], sem),
                pltpu.async_copy(rel_hbm.at[r_idx.at[sl]],
                                 r_rows.at[sl], sem),
                pltpu.async_copy(ent_hbm.at[t_idx.at[sl]],
                                 t_rows.at[sl], sem),
            ))

        def body(i, acc):
            vv = v_vm[i, :]
            dd = jnp.zeros((_LANES,), jnp.float32)
            for c in range(_CHUNKS):
                sl = pl.ds(c * _LANES, _LANES)
                d = h_rows[i, sl] + r_rows[i, sl] - t_rows[i, sl]
                dd = dd + d * d
            return acc + dd * vv

        cp_v.wait()
        acc = jnp.zeros((_LANES,), jnp.float32)
        for c in range(2):
            for cp in halves[c]:
                cp.wait()
            acc = lax.fori_loop(c * half, (c + 1) * half, body, acc)

        acc_vm[...] = acc
        pltpu.sync_copy(acc_vm, out_hbm.at[wid])

    return loss_kernel


def kernel(h, r, t, v, adj, ent_emb, rel_emb, W, b):
    info = plsc.get_sparse_core_info()
    num_workers = info.num_cores * info.num_subcores
    b_per_w = _B // num_workers
    loss_kernel = _make_loss_kernel(num_workers, b_per_w)
    v_rep = jnp.broadcast_to(v.astype(jnp.float32)[:, None], (_B, _LANES))
    partials = loss_kernel(
        h.astype(jnp.int32), r.astype(jnp.int32), t.astype(jnp.int32),
        v_rep, ent_emb, rel_emb)
    return jnp.sum(partials) / jnp.float32(_B)
